# Initial kernel scaffold; baseline (speedup 1.0000x reference)
import jax
import jax.numpy as jnp
from jax.experimental import pallas as pl


def kernel(discrete_x, continous_x, edge_index, edge_attr, churn_date, t, params):
    raise NotImplementedError("write your pallas kernel here")


# speedup vs baseline: 4.5071x; 4.5071x over previous
import numpy as np
import jax
import jax.numpy as jnp
from jax.experimental import pallas as pl

_N = 50000
_E = 800000
_ND = 38
_NC = 16
_CC = 8
_NH = 64
_DE = 16
_H = 4
_NH0 = _ND - 6 + 3 * _CC
_NH1 = _NH0 + _NH

_BLK_N = 5000
_BLK_E = 4000


def _leaky(x, s):
    return jnp.where(x >= 0, x, x * s)


def _rowmap(body, n_rows, blk, row_args, full_args, out_dims):
    def _row_spec(a):
        nd = a.ndim
        return pl.BlockSpec((blk,) + a.shape[1:],
                            lambda i, _nd=nd: (i,) + (0,) * (_nd - 1))

    def _full_spec(a):
        nd = a.ndim
        return pl.BlockSpec(a.shape, lambda i, _nd=nd: (0,) * _nd)

    in_specs = [_row_spec(a) for a in row_args] + [_full_spec(a) for a in full_args]
    out_specs = [pl.BlockSpec((blk, d), lambda i: (i, 0)) for d in out_dims]
    out_shape = [jax.ShapeDtypeStruct((n_rows, d), jnp.float32) for d in out_dims]
    res = pl.pallas_call(
        body,
        grid=(n_rows // blk,),
        in_specs=in_specs,
        out_specs=out_specs,
        out_shape=out_shape,
    )(*row_args, *full_args)
    return res


def _k_dinv(deg_ref, o_ref):
    o_ref[...] = jax.lax.rsqrt(jnp.maximum(deg_ref[...], 1.0))


def _k_embed(d_ref, c_ref, Sd, Wcb, bcb, Sc, Wg, bg, W1, xdc_o, h1_o):
    d = d_ref[...]
    c = c_ref[...]
    xc = _leaky(c @ Wcb[...] + bcb[...], 0.01)
    xdc = d @ Sd[...] + xc @ Sc[...]
    xg0 = _leaky(xdc @ Wg[...] + bg[...], 0.01)
    xdc_o[...] = xdc
    h1_o[...] = xg0 @ W1[...]


def _k_gcn_msg(hs_ref, ds_ref, dd_ref, o_ref):
    o_ref[...] = hs_ref[...] * (ds_ref[...] * dd_ref[...])


def _k_gcn_mid(seg_ref, h_ref, dinv_ref, b1, W2, o_ref):
    dv = dinv_ref[...]
    o1 = seg_ref[...] + h_ref[...] * (dv * dv) + b1[...]
    o_ref[...] = _leaky(o1, 0.01) @ W2[...]


def _k_dense(seg_ref, h2_ref, dinv_ref, xdc_ref,
             b2, Pa, Pb, r1W, r1b, r2W, r2b, cw0, cb0, cw1, cb1,
             WfT, WfB, fb, siW, sib, We1, As1, Ad1,
             hci_o, hm_o, as_o, ad_o):
    dv = dinv_ref[...]
    o2 = seg_ref[...] + h2_ref[...] * (dv * dv) + b2[...]
    xg = _leaky(o2, 0.01)
    x = xdc_ref[...] @ Pa[...] + xg @ Pb[...]
    h = x + _leaky(x @ r1W[...] + r1b[...], 0.01)
    h = h + _leaky(h @ r2W[...] + r2b[...], 0.01)
    s0 = jnp.sum(x * cw0[...], axis=1, keepdims=True)
    xl = x * s0 + cb0[...] + x
    s1 = jnp.sum(xl * cw1[...], axis=1, keepdims=True)
    xl = x * s1 + cb1[...] + xl
    hci = _leaky(h @ WfT[...] + xl @ WfB[...] + fb[...], 0.01)
    xsi = _leaky(x @ siW[...] + sib[...], 0.01)
    hm = xsi @ We1[...]
    hci_o[...] = hci
    hm_o[...] = hm
    as_o[...] = hm @ As1[...]
    ad_o[...] = hm @ Ad1[...]


def _k_edge(ea_ref, e1W, e1b, e2W, e2b, WAe1, WAe2, ae1_o, ae2_o):
    ea = jnp.maximum(ea_ref[...] @ e1W[...] + e1b[...], 0.0)
    ea = jnp.maximum(ea @ e2W[...] + e2b[...], 0.0)
    ae1_o[...] = ea @ WAe1[...]
    ae2_o[...] = ea @ WAe2[...]


def _k_logits(as_ref, ad_ref, ae_ref, o_ref):
    o_ref[...] = _leaky(as_ref[...] + ad_ref[...] + ae_ref[...], 0.2)


def _k_exp(lg_ref, md_ref, o_ref):
    o_ref[...] = jnp.exp(lg_ref[...] - md_ref[...])


def _k_attmsg(ex_ref, dend_ref, hms_ref, R, o_ref):
    att = ex_ref[...] / (dend_ref[...] + 1e-16)
    o_ref[...] = hms_ref[...] * (att @ R[...])


def _k_egat_mid(out_ref, Mh, be, We2, As2, Ad2, hm_o, as_o, ad_o):
    xs = _leaky(out_ref[...] @ Mh[...] + be[...], 0.01)
    hm = xs @ We2[...]
    hm_o[...] = hm
    as_o[...] = hm @ As2[...]
    ad_o[...] = hm @ Ad2[...]


def _k_egat_out(out_ref, Mh, be2, l1W, l1b, o_ref):
    xs = _leaky(out_ref[...] @ Mh[...] + be2[...], 0.01)
    o_ref[...] = _leaky(xs @ l1W[...] + l1b[...], 0.01)


def _k_heads(hci_ref, hsi_ref, tf_ref,
             TW, Tb, A0T, A0B, a0b, A1T, A1B, a1b, S0, S1,
             w01, b01, w02, b02, w03, b03,
             w11, b11, w12, b12, w13, b13,
             py_o, cf_o, p0_o, p1_o, pt_o):
    hci = hci_ref[...]
    hsi = hsi_ref[...]
    pt_o[...] = hsi @ TW[...] + Tb[...]

    def _softmax(z):
        z = z - jnp.max(z, axis=1, keepdims=True)
        e = jnp.exp(z)
        return e / jnp.sum(e, axis=1, keepdims=True)

    a0 = _softmax(hci @ A0T[...] + hsi @ A0B[...] + a0b[...])
    py0 = (a0 @ S0[...]) * hci + (a0 @ S1[...]) * hsi
    a1 = _softmax(hci @ A1T[...] + hsi @ A1B[...] + a1b[...])
    py1 = (a1 @ S0[...]) * hci + (a1 @ S1[...]) * hsi

    def _pred(z, w1, b1, w2, b2, w3, b3):
        z = _leaky(z @ w1[...] + b1[...], 0.01)
        z = _leaky(z @ w2[...] + b2[...], 0.01)
        return z @ w3[...] + b3[...]

    p0 = _pred(py0, w01, b01, w02, b02, w03, b03)
    p1 = _pred(py1, w11, b11, w12, b12, w13, b13)
    tf = tf_ref[...]
    py_o[...] = (1.0 - tf) * p0 + tf * p1
    cf_o[...] = tf * p0 + (1.0 - tf) * p1
    p0_o[...] = p0
    p1_o[...] = p1


def _row(v):
    return v.reshape(1, -1)


def _head_mix(a):
    return (jnp.eye(_H, dtype=jnp.float32)[:, None, :]
            * a[:, :, None]).reshape(_H * _NH, _H)


def kernel(discrete_x, continous_x, edge_index, edge_attr, churn_date, t, params):
    del churn_date
    f32 = jnp.float32
    src, dst = edge_index[0], edge_index[1]
    L = t.shape[0]

    Sd = np.zeros((_ND, _NH0), np.float32)
    for j in range(_ND - 6):
        Sd[6 + j, j] = 1.0
    Sc = np.zeros((3 * _CC, _NH0), np.float32)
    for j in range(3 * _CC):
        Sc[j, _ND - 6 + j] = 1.0
    Pa = np.zeros((_NH0, _NH1), np.float32)
    Pa[np.arange(_NH0), np.arange(_NH0)] = 1.0
    Pb = np.zeros((_NH, _NH1), np.float32)
    Pb[np.arange(_NH), _NH0 + np.arange(_NH)] = 1.0
    S0 = np.zeros((2 * _NH, _NH), np.float32)
    S0[np.arange(_NH), np.arange(_NH)] = 1.0
    S1 = np.zeros((2 * _NH, _NH), np.float32)
    S1[_NH + np.arange(_NH), np.arange(_NH)] = 1.0
    Sd, Sc, Pa, Pb, S0, S1 = map(jnp.asarray, (Sd, Sc, Pa, Pb, S0, S1))
    R = jnp.asarray(np.repeat(np.eye(_H, dtype=np.float32), _NH, axis=1))
    Mh = jnp.asarray(np.tile(np.eye(_NH, dtype=np.float32), (_H, 1)) * (1.0 / _H))

    Wc = params["emb_c"]["W"]
    Wcb = jnp.zeros((3 * _NC, 3 * _CC), f32)
    for k in range(3):
        Wcb = Wcb.at[k * _NC:(k + 1) * _NC, k * _CC:(k + 1) * _CC].set(Wc)
    bcb = _row(jnp.tile(params["emb_c"]["b"], 3))

    e1, e2 = params["egat1"], params["egat2"]
    As1, Ad1 = _head_mix(e1["a_src"]), _head_mix(e1["a_dst"])
    As2, Ad2 = _head_mix(e2["a_src"]), _head_mix(e2["a_dst"])
    WAe1 = e1["We"] @ _head_mix(e1["a_e"])
    WAe2 = e2["We"] @ _head_mix(e2["a_e"])
    Wf = params["fusion"]["W"]
    WfT, WfB = Wf[:_NH1], Wf[_NH1:]
    ay0W, ay1W = params["ay0"]["W"], params["ay1"]["W"]
    A0T, A0B = ay0W[:_NH], ay0W[_NH:]
    A1T, A1B = ay1W[:_NH], ay1W[_NH:]

    deg = (jnp.zeros((_N,), f32).at[dst].add(1.0) + 1.0).reshape(_N, 1)
    dinv, = _rowmap(_k_dinv, _N, _BLK_N, [deg], [], [1])

    xdc, h1 = _rowmap(
        _k_embed, _N, _BLK_N, [discrete_x, continous_x],
        [Sd, Wcb, bcb, Sc, params["g0"]["W"], _row(params["g0"]["b"]),
         params["gcn1"]["W"]],
        [_NH0, _NH])

    msg1, = _rowmap(_k_gcn_msg, _E, _BLK_E, [h1[src], dinv[src], dinv[dst]], [], [_NH])
    seg1 = jnp.zeros((_N, _NH), f32).at[dst].add(msg1)
    h2, = _rowmap(_k_gcn_mid, _N, _BLK_N, [seg1, h1, dinv],
                  [_row(params["gcn1"]["b"]), params["gcn2"]["W"]], [_NH])

    msg2, = _rowmap(_k_gcn_msg, _E, _BLK_E, [h2[src], dinv[src], dinv[dst]], [], [_NH])
    seg2 = jnp.zeros((_N, _NH), f32).at[dst].add(msg2)

    hci, hm1, as1, ad1 = _rowmap(
        _k_dense, _N, _BLK_N, [seg2, h2, dinv, xdc],
        [_row(params["gcn2"]["b"]), Pa, Pb,
         params["res1"]["W"], _row(params["res1"]["b"]),
         params["res2"]["W"], _row(params["res2"]["b"]),
         _row(params["cross_w"][0]), _row(params["cross_b"][0]),
         _row(params["cross_w"][1]), _row(params["cross_b"][1]),
         WfT, WfB, _row(params["fusion"]["b"]),
         params["si0"]["W"], _row(params["si0"]["b"]),
         e1["W"], As1, Ad1],
        [_NH, _H * _NH, _H, _H])

    ae1, ae2 = _rowmap(
        _k_edge, _E, _BLK_E, [edge_attr],
        [params["e1"]["W"], _row(params["e1"]["b"]),
         params["e2"]["W"], _row(params["e2"]["b"]), WAe1, WAe2],
        [_H, _H])

    def _egat_pass(asn, adn, ae, hm):
        lg, = _rowmap(_k_logits, _E, _BLK_E, [asn[src], adn[dst], ae], [], [_H])
        m = jnp.full((_N, _H), -1e30, f32).at[dst].max(lg)
        ex, = _rowmap(_k_exp, _E, _BLK_E, [lg, m[dst]], [], [_H])
        den = jnp.zeros((_N, _H), f32).at[dst].add(ex)
        msgs, = _rowmap(_k_attmsg, _E, _BLK_E, [ex, den[dst], hm[src]], [R],
                        [_H * _NH])
        return jnp.zeros((_N, _H * _NH), f32).at[dst].add(msgs)

    out1 = _egat_pass(as1, ad1, ae1, hm1)
    hm2, as2, ad2 = _rowmap(
        _k_egat_mid, _N, _BLK_N, [out1],
        [Mh, _row(e1["b"]), e2["W"], As2, Ad2],
        [_H * _NH, _H, _H])
    out2 = _egat_pass(as2, ad2, ae2, hm2)
    hsi, = _rowmap(
        _k_egat_out, _N, _BLK_N, [out2],
        [Mh, _row(e2["b"]), params["lin1"]["W"], _row(params["lin1"]["b"])],
        [_NH])

    tf = t.astype(f32)
    p = params
    pred_y, pred_y_cf, py0, py1, pred_T = _rowmap(
        _k_heads, L, _BLK_N, [hci[:L], hsi[:L], tf],
        [p["T"]["W"], _row(p["T"]["b"]),
         A0T, A0B, _row(p["ay0"]["b"]), A1T, A1B, _row(p["ay1"]["b"]), S0, S1,
         p["y0_1"]["W"], _row(p["y0_1"]["b"]), p["y0_2"]["W"], _row(p["y0_2"]["b"]),
         p["y0_3"]["W"], _row(p["y0_3"]["b"]),
         p["y1_1"]["W"], _row(p["y1_1"]["b"]), p["y1_2"]["W"], _row(p["y1_2"]["b"]),
         p["y1_3"]["W"], _row(p["y1_3"]["b"])],
        [1, 1, 1, 1, 1])

    return (pred_y, pred_y_cf, py0, py1, pred_T, hci, hsi)


# speedup vs baseline: 4.8287x; 1.0714x over previous
import numpy as np
import jax
import jax.numpy as jnp
from jax.experimental import pallas as pl

_N = 50000
_E = 800000
_ND = 38
_NC = 16
_CC = 8
_NH = 64
_DE = 16
_H = 4
_NH0 = _ND - 6 + 3 * _CC
_NH1 = _NH0 + _NH

_BLK_N = 5000
_BLK_E = 4000


def _leaky(x, s):
    return jnp.where(x >= 0, x, x * s)


def _rowmap(body, n_rows, blk, row_args, full_args, out_dims):
    def _row_spec(a):
        nd = a.ndim
        return pl.BlockSpec((blk,) + a.shape[1:],
                            lambda i, _nd=nd: (i,) + (0,) * (_nd - 1))

    def _full_spec(a):
        nd = a.ndim
        return pl.BlockSpec(a.shape, lambda i, _nd=nd: (0,) * _nd)

    in_specs = [_row_spec(a) for a in row_args] + [_full_spec(a) for a in full_args]
    out_specs = [pl.BlockSpec((blk, d), lambda i: (i, 0)) for d in out_dims]
    out_shape = [jax.ShapeDtypeStruct((n_rows, d), jnp.float32) for d in out_dims]
    res = pl.pallas_call(
        body,
        grid=(n_rows // blk,),
        in_specs=in_specs,
        out_specs=out_specs,
        out_shape=out_shape,
    )(*row_args, *full_args)
    return res


def _k_dinv(deg_ref, o_ref):
    o_ref[...] = jax.lax.rsqrt(jnp.maximum(deg_ref[...], 1.0))


def _k_embed(d_ref, c_ref, Sd, Wcb, bcb, Sc, Wg, bg, W1, xdc_o, h1_o):
    d = d_ref[...]
    c = c_ref[...]
    xc = _leaky(c @ Wcb[...] + bcb[...], 0.01)
    xdc = d @ Sd[...] + xc @ Sc[...]
    xg0 = _leaky(xdc @ Wg[...] + bg[...], 0.01)
    xdc_o[...] = xdc
    h1_o[...] = xg0 @ W1[...]


def _k_gcn_msg(hs_ref, ds_ref, dd_ref, o_ref):
    o_ref[...] = hs_ref[...] * (ds_ref[...] * dd_ref[...])


def _k_gcn_mid(seg_ref, h_ref, dinv_ref, b1, W2, o_ref):
    dv = dinv_ref[...]
    o1 = seg_ref[...] + h_ref[...] * (dv * dv) + b1[...]
    o_ref[...] = _leaky(o1, 0.01) @ W2[...]


def _k_dense(seg_ref, h2_ref, dinv_ref, xdc_ref,
             b2, Pa, Pb, r1W, r1b, r2W, r2b, cw0, cb0, cw1, cb1,
             WfT, WfB, fb, siW, sib, We1, As1, Ad1,
             hci_o, hm_o, as_o, ad_o):
    dv = dinv_ref[...]
    o2 = seg_ref[...] + h2_ref[...] * (dv * dv) + b2[...]
    xg = _leaky(o2, 0.01)
    x = xdc_ref[...] @ Pa[...] + xg @ Pb[...]
    h = x + _leaky(x @ r1W[...] + r1b[...], 0.01)
    h = h + _leaky(h @ r2W[...] + r2b[...], 0.01)
    s0 = jnp.sum(x * cw0[...], axis=1, keepdims=True)
    xl = x * s0 + cb0[...] + x
    s1 = jnp.sum(xl * cw1[...], axis=1, keepdims=True)
    xl = x * s1 + cb1[...] + xl
    hci = _leaky(h @ WfT[...] + xl @ WfB[...] + fb[...], 0.01)
    xsi = _leaky(x @ siW[...] + sib[...], 0.01)
    hm = xsi @ We1[...]
    hci_o[...] = hci
    hm_o[...] = hm
    as_o[...] = hm @ As1[...]
    ad_o[...] = hm @ Ad1[...]


def _k_edge(ea_ref, e1W, e1b, e2W, e2b, WAe1, WAe2, ae1_o, ae2_o):
    ea = jnp.maximum(ea_ref[...] @ e1W[...] + e1b[...], 0.0)
    ea = jnp.maximum(ea @ e2W[...] + e2b[...], 0.0)
    ae1_o[...] = ea @ WAe1[...]
    ae2_o[...] = ea @ WAe2[...]


def _k_logits(as_ref, ad_ref, ae_ref, o_ref):
    o_ref[...] = _leaky(as_ref[...] + ad_ref[...] + ae_ref[...], 0.2)


def _k_exp(lg_ref, md_ref, o_ref):
    o_ref[...] = jnp.exp(lg_ref[...] - md_ref[...])


def _k_attmsg(ex_ref, dend_ref, hms_ref, R, Mh, o_ref):
    att = ex_ref[...] / (dend_ref[...] + 1e-16)
    o_ref[...] = (hms_ref[...] * (att @ R[...])) @ Mh[...]


def _k_egat_mid(out_ref, be, We2, As2, Ad2, hm_o, as_o, ad_o):
    xs = _leaky(out_ref[...] + be[...], 0.01)
    hm = xs @ We2[...]
    hm_o[...] = hm
    as_o[...] = hm @ As2[...]
    ad_o[...] = hm @ Ad2[...]


def _k_egat_out(out_ref, be2, l1W, l1b, o_ref):
    xs = _leaky(out_ref[...] + be2[...], 0.01)
    o_ref[...] = _leaky(xs @ l1W[...] + l1b[...], 0.01)


def _k_heads(hci_ref, hsi_ref, tf_ref,
             TW, Tb, A0T, A0B, a0b, A1T, A1B, a1b, S0, S1,
             w01, b01, w02, b02, w03, b03,
             w11, b11, w12, b12, w13, b13,
             py_o, cf_o, p0_o, p1_o, pt_o):
    hci = hci_ref[...]
    hsi = hsi_ref[...]
    pt_o[...] = hsi @ TW[...] + Tb[...]

    def _softmax(z):
        z = z - jnp.max(z, axis=1, keepdims=True)
        e = jnp.exp(z)
        return e / jnp.sum(e, axis=1, keepdims=True)

    a0 = _softmax(hci @ A0T[...] + hsi @ A0B[...] + a0b[...])
    py0 = (a0 @ S0[...]) * hci + (a0 @ S1[...]) * hsi
    a1 = _softmax(hci @ A1T[...] + hsi @ A1B[...] + a1b[...])
    py1 = (a1 @ S0[...]) * hci + (a1 @ S1[...]) * hsi

    def _pred(z, w1, b1, w2, b2, w3, b3):
        z = _leaky(z @ w1[...] + b1[...], 0.01)
        z = _leaky(z @ w2[...] + b2[...], 0.01)
        return z @ w3[...] + b3[...]

    p0 = _pred(py0, w01, b01, w02, b02, w03, b03)
    p1 = _pred(py1, w11, b11, w12, b12, w13, b13)
    tf = tf_ref[...]
    py_o[...] = (1.0 - tf) * p0 + tf * p1
    cf_o[...] = tf * p0 + (1.0 - tf) * p1
    p0_o[...] = p0
    p1_o[...] = p1


def _row(v):
    return v.reshape(1, -1)


def _head_mix(a):
    return (jnp.eye(_H, dtype=jnp.float32)[:, None, :]
            * a[:, :, None]).reshape(_H * _NH, _H)


def kernel(discrete_x, continous_x, edge_index, edge_attr, churn_date, t, params):
    del churn_date
    f32 = jnp.float32
    src, dst = edge_index[0], edge_index[1]
    L = t.shape[0]

    Sd = np.zeros((_ND, _NH0), np.float32)
    for j in range(_ND - 6):
        Sd[6 + j, j] = 1.0
    Sc = np.zeros((3 * _CC, _NH0), np.float32)
    for j in range(3 * _CC):
        Sc[j, _ND - 6 + j] = 1.0
    Pa = np.zeros((_NH0, _NH1), np.float32)
    Pa[np.arange(_NH0), np.arange(_NH0)] = 1.0
    Pb = np.zeros((_NH, _NH1), np.float32)
    Pb[np.arange(_NH), _NH0 + np.arange(_NH)] = 1.0
    S0 = np.zeros((2 * _NH, _NH), np.float32)
    S0[np.arange(_NH), np.arange(_NH)] = 1.0
    S1 = np.zeros((2 * _NH, _NH), np.float32)
    S1[_NH + np.arange(_NH), np.arange(_NH)] = 1.0
    Sd, Sc, Pa, Pb, S0, S1 = map(jnp.asarray, (Sd, Sc, Pa, Pb, S0, S1))
    R = jnp.asarray(np.repeat(np.eye(_H, dtype=np.float32), _NH, axis=1))
    Mh = jnp.asarray(np.tile(np.eye(_NH, dtype=np.float32), (_H, 1)) * (1.0 / _H))

    Wc = params["emb_c"]["W"]
    Wcb = jnp.zeros((3 * _NC, 3 * _CC), f32)
    for k in range(3):
        Wcb = Wcb.at[k * _NC:(k + 1) * _NC, k * _CC:(k + 1) * _CC].set(Wc)
    bcb = _row(jnp.tile(params["emb_c"]["b"], 3))

    e1, e2 = params["egat1"], params["egat2"]
    As1, Ad1 = _head_mix(e1["a_src"]), _head_mix(e1["a_dst"])
    As2, Ad2 = _head_mix(e2["a_src"]), _head_mix(e2["a_dst"])
    WAe1 = e1["We"] @ _head_mix(e1["a_e"])
    WAe2 = e2["We"] @ _head_mix(e2["a_e"])
    Wf = params["fusion"]["W"]
    WfT, WfB = Wf[:_NH1], Wf[_NH1:]
    ay0W, ay1W = params["ay0"]["W"], params["ay1"]["W"]
    A0T, A0B = ay0W[:_NH], ay0W[_NH:]
    A1T, A1B = ay1W[:_NH], ay1W[_NH:]

    deg = (jnp.zeros((_N,), f32).at[dst].add(1.0) + 1.0).reshape(_N, 1)
    dinv, = _rowmap(_k_dinv, _N, _BLK_N, [deg], [], [1])

    xdc, h1 = _rowmap(
        _k_embed, _N, _BLK_N, [discrete_x, continous_x],
        [Sd, Wcb, bcb, Sc, params["g0"]["W"], _row(params["g0"]["b"]),
         params["gcn1"]["W"]],
        [_NH0, _NH])

    msg1, = _rowmap(_k_gcn_msg, _E, _BLK_E, [h1[src], dinv[src], dinv[dst]], [], [_NH])
    seg1 = jnp.zeros((_N, _NH), f32).at[dst].add(msg1)
    h2, = _rowmap(_k_gcn_mid, _N, _BLK_N, [seg1, h1, dinv],
                  [_row(params["gcn1"]["b"]), params["gcn2"]["W"]], [_NH])

    msg2, = _rowmap(_k_gcn_msg, _E, _BLK_E, [h2[src], dinv[src], dinv[dst]], [], [_NH])
    seg2 = jnp.zeros((_N, _NH), f32).at[dst].add(msg2)

    hci, hm1, as1, ad1 = _rowmap(
        _k_dense, _N, _BLK_N, [seg2, h2, dinv, xdc],
        [_row(params["gcn2"]["b"]), Pa, Pb,
         params["res1"]["W"], _row(params["res1"]["b"]),
         params["res2"]["W"], _row(params["res2"]["b"]),
         _row(params["cross_w"][0]), _row(params["cross_b"][0]),
         _row(params["cross_w"][1]), _row(params["cross_b"][1]),
         WfT, WfB, _row(params["fusion"]["b"]),
         params["si0"]["W"], _row(params["si0"]["b"]),
         e1["W"], As1, Ad1],
        [_NH, _H * _NH, _H, _H])

    ae1, ae2 = _rowmap(
        _k_edge, _E, _BLK_E, [edge_attr],
        [params["e1"]["W"], _row(params["e1"]["b"]),
         params["e2"]["W"], _row(params["e2"]["b"]), WAe1, WAe2],
        [_H, _H])

    def _egat_pass(asn, adn, ae, hm):
        lg, = _rowmap(_k_logits, _E, _BLK_E, [asn[src], adn[dst], ae], [], [_H])
        m = jnp.full((_N, _H), -1e30, f32).at[dst].max(lg)
        ex, = _rowmap(_k_exp, _E, _BLK_E, [lg, m[dst]], [], [_H])
        den = jnp.zeros((_N, _H), f32).at[dst].add(ex)
        msgs, = _rowmap(_k_attmsg, _E, _BLK_E, [ex, den[dst], hm[src]], [R, Mh],
                        [_NH])
        return jnp.zeros((_N, _NH), f32).at[dst].add(msgs)

    out1 = _egat_pass(as1, ad1, ae1, hm1)
    hm2, as2, ad2 = _rowmap(
        _k_egat_mid, _N, _BLK_N, [out1],
        [_row(e1["b"]), e2["W"], As2, Ad2],
        [_H * _NH, _H, _H])
    out2 = _egat_pass(as2, ad2, ae2, hm2)
    hsi, = _rowmap(
        _k_egat_out, _N, _BLK_N, [out2],
        [_row(e2["b"]), params["lin1"]["W"], _row(params["lin1"]["b"])],
        [_NH])

    tf = t.astype(f32)
    p = params
    pred_y, pred_y_cf, py0, py1, pred_T = _rowmap(
        _k_heads, L, _BLK_N, [hci[:L], hsi[:L], tf],
        [p["T"]["W"], _row(p["T"]["b"]),
         A0T, A0B, _row(p["ay0"]["b"]), A1T, A1B, _row(p["ay1"]["b"]), S0, S1,
         p["y0_1"]["W"], _row(p["y0_1"]["b"]), p["y0_2"]["W"], _row(p["y0_2"]["b"]),
         p["y0_3"]["W"], _row(p["y0_3"]["b"]),
         p["y1_1"]["W"], _row(p["y1_1"]["b"]), p["y1_2"]["W"], _row(p["y1_2"]["b"]),
         p["y1_3"]["W"], _row(p["y1_3"]["b"])],
        [1, 1, 1, 1, 1])

    return (pred_y, pred_y_cf, py0, py1, pred_T, hci, hsi)
